# baseline (speedup 1.0000x reference)
import jax
import jax.numpy as jnp
from jax import lax
from jax.experimental import pallas as pl
from jax.experimental.pallas import tpu as pltpu
from jax.experimental.pallas import tpu_sc as plsc

N = 10000
D = 128
F2 = 64
LAT = 32
E = 320000
NC, NS = 2, 16
CH = 125
NCHT = E // CH
NCH = NCHT // NS
NACC = 10240
RPT = NACC // NS
ZR = 128
NBUF = 5
NWK = NC * NS
DEGC = NCHT // NWK
RB = N
GRID = N // RB


def _sc_mesh():
    return plsc.VectorSubcoreMesh(core_axis_name="c", subcore_axis_name="s")


def _deg_body(sd_hbm, out_hbm, dstv, onev, zbuf, acc, sem):
    c = lax.axis_index("c")
    s = lax.axis_index("s")
    wid = s * NC + c
    pltpu.sync_copy(sd_hbm.at[1, pl.ds(wid * DEGC, DEGC)], dstv)

    for i in range(7):
        onev[pl.ds(i * 16, 16)] = jnp.ones((16,), jnp.float32)
    onev[pl.ds(CH - 16, 16)] = jnp.ones((16,), jnp.float32)

    def _setzero(i, carry):
        zbuf[pl.ds(i * 16, 16)] = jnp.zeros((16,), jnp.float32)
        return carry

    lax.fori_loop(0, RPT // 16, _setzero, 0)
    pltpu.sync_copy(zbuf, acc.at[pl.ds(s * RPT, RPT)])
    plsc.subcore_barrier()

    def _chunk(j, carry):
        pltpu.async_copy(onev, acc.at[dstv.at[j]], sem, add=True)
        return carry

    lax.fori_loop(0, DEGC, _chunk, 0)

    def _drain(j, carry):
        pltpu.make_async_copy(onev, acc.at[dstv.at[0]], sem).wait()
        return carry

    lax.fori_loop(0, DEGC, _drain, 0)
    plsc.subcore_barrier()
    pltpu.sync_copy(acc.at[pl.ds(s * RPT, RPT)], out_hbm.at[c, pl.ds(s * RPT, RPT)])


def _deg_call(sd):
    f = pl.kernel(
        _deg_body,
        out_type=pltpu.HBM((NC, NACC), jnp.float32),
        mesh=_sc_mesh(),
        scratch_types=[
            pltpu.VMEM((DEGC, CH), jnp.int32),
            pltpu.VMEM((CH,), jnp.float32),
            pltpu.VMEM((RPT,), jnp.float32),
            pltpu.VMEM_SHARED((NACC,), jnp.float32),
            pltpu.SemaphoreType.DMA,
        ],
    )
    return f(sd)


def _make_scatter_body(H, from_spmem):
    def body(g_hbm, sd_hbm, out_hbm,
             srcv, dstv, rows, zbuf, acc, *rest):
        if from_spmem:
            gsp = rest[0]
            rest = rest[1:]
        gsem = list(rest[:NBUF])
        ssem = list(rest[NBUF:])
        c = lax.axis_index("c")
        s = lax.axis_index("s")
        pltpu.sync_copy(sd_hbm.at[0, pl.ds(s * NCH, NCH)], srcv)
        pltpu.sync_copy(sd_hbm.at[1, pl.ds(s * NCH, NCH)], dstv)

        if from_spmem:
            rpg = N // NS
            pltpu.sync_copy(g_hbm.at[c, pl.ds(s * rpg, rpg)],
                            gsp.at[pl.ds(s * rpg, rpg)])
            gref = gsp
        else:
            gref = g_hbm.at[c]

        K = NBUF // 2
        if not from_spmem:
            for b in range(K):
                pltpu.async_copy(gref.at[srcv.at[b]], rows.at[b], gsem[b])

        def _zrow(i, carry):
            for k in range(H // 16):
                zbuf[i, pl.ds(k * 16, 16)] = jnp.zeros((16,), jnp.float32)
            return carry

        lax.fori_loop(0, ZR, _zrow, 0)
        for r in range(RPT // ZR):
            pltpu.sync_copy(zbuf, acc.at[pl.ds(s * RPT + r * ZR, ZR)])
        plsc.subcore_barrier()

        for b in range(0 if from_spmem else K, NBUF):
            pltpu.async_copy(gref.at[srcv.at[b]], rows.at[b], gsem[b])

        def outer(j0, carry):
            for b in range(NBUF):
                j = j0 + b
                pltpu.make_async_copy(gref.at[srcv.at[0]], rows.at[b],
                                      gsem[b]).wait()
                pltpu.async_copy(rows.at[b], acc.at[dstv.at[j]],
                                 ssem[b], add=True)
            for b in range(NBUF):
                j = j0 + b
                pltpu.make_async_copy(rows.at[b], acc.at[dstv.at[0]],
                                      ssem[b]).wait()

                @pl.when(j + NBUF < NCH)
                def _fire(b=b, j=j):
                    pltpu.async_copy(gref.at[srcv.at[j + NBUF]],
                                     rows.at[b], gsem[b])
            return carry

        lax.fori_loop(0, NCH // NBUF, lambda i, cr: outer(i * NBUF, cr), 0)
        plsc.subcore_barrier()
        pltpu.sync_copy(acc.at[pl.ds(s * RPT, RPT)],
                        out_hbm.at[c, pl.ds(s * RPT, RPT)])

    return body


def _scatter_call(H, gs, sd, from_spmem=False):
    f = pl.kernel(
        _make_scatter_body(H, from_spmem),
        out_type=pltpu.HBM((NC, NACC, H), jnp.float32),
        mesh=_sc_mesh(),
        compiler_params=pltpu.CompilerParams(use_tc_tiling_on_sc=False),
        scratch_types=(
            [pltpu.VMEM((NCH, CH), jnp.int32),
             pltpu.VMEM((NCH, CH), jnp.int32),
             pltpu.VMEM((NBUF, CH, H), jnp.float32),
             pltpu.VMEM((ZR, H), jnp.float32),
             pltpu.VMEM_SHARED((NACC, H), jnp.float32)]
            + ([pltpu.VMEM_SHARED((N, H), jnp.float32)] if from_spmem else [])
            + [pltpu.SemaphoreType.DMA] * (2 * NBUF)
        ),
    )
    return f(gs, sd)


def _tc1_body(x_ref, wa_ref, wb_ref, d0_ref, d1_ref, gs_ref):
    dinv = lax.rsqrt(d0_ref[...] + d1_ref[...] + 1.0)
    x = x_ref[...]
    gs_ref[0] = jnp.dot(x, wa_ref[...],
                        preferred_element_type=jnp.float32) * dinv
    gs_ref[1] = jnp.dot(x, wb_ref[...],
                        preferred_element_type=jnp.float32) * dinv


def _tc1_call(x, W1a, W1b, d0, d1):
    return pl.pallas_call(
        _tc1_body,
        grid=(GRID,),
        in_specs=[
            pl.BlockSpec((RB, D), lambda i: (i, 0)),
            pl.BlockSpec((D, D // 2), lambda i: (0, 0)),
            pl.BlockSpec((D, D // 2), lambda i: (0, 0)),
            pl.BlockSpec((RB, 1), lambda i: (i, 0)),
            pl.BlockSpec((RB, 1), lambda i: (i, 0)),
        ],
        out_specs=pl.BlockSpec((NC, RB, D // 2), lambda i: (0, i, 0)),
        out_shape=jax.ShapeDtypeStruct((NC, N, D // 2), jnp.float32),
    )(x, W1a, W1b, d0, d1)


def _tc2_body(s1_ref, gs_ref, d0_ref, d1_ref, wmua_ref, wmub_ref,
              wlva_ref, wlvb_ref, b1a_ref, b1b_ref, g1s_ref):
    dinv = lax.rsqrt(d0_ref[...] + d1_ref[...] + 1.0)
    ha = jnp.maximum((s1_ref[0] + gs_ref[0]) * dinv + b1a_ref[...], 0.0)
    hb = jnp.maximum((s1_ref[1] + gs_ref[1]) * dinv + b1b_ref[...], 0.0)
    mm = jnp.dot(ha, wmua_ref[...], preferred_element_type=jnp.float32)
    mm += jnp.dot(hb, wmub_ref[...], preferred_element_type=jnp.float32)
    lv = jnp.dot(ha, wlva_ref[...], preferred_element_type=jnp.float32)
    lv += jnp.dot(hb, wlvb_ref[...], preferred_element_type=jnp.float32)
    g1s_ref[0] = mm * dinv
    g1s_ref[1] = lv * dinv


def _tc2_call(s1, gs, d0, d1, Wmua, Wmub, Wlva, Wlvb, b1a, b1b):
    return pl.pallas_call(
        _tc2_body,
        grid=(GRID,),
        in_specs=[
            pl.BlockSpec((NC, RB, D // 2), lambda i: (0, i, 0)),
            pl.BlockSpec((NC, RB, D // 2), lambda i: (0, i, 0)),
            pl.BlockSpec((RB, 1), lambda i: (i, 0)),
            pl.BlockSpec((RB, 1), lambda i: (i, 0)),
            pl.BlockSpec((D // 2, LAT), lambda i: (0, 0)),
            pl.BlockSpec((D // 2, LAT), lambda i: (0, 0)),
            pl.BlockSpec((D // 2, LAT), lambda i: (0, 0)),
            pl.BlockSpec((D // 2, LAT), lambda i: (0, 0)),
            pl.BlockSpec((1, D // 2), lambda i: (0, 0)),
            pl.BlockSpec((1, D // 2), lambda i: (0, 0)),
        ],
        out_specs=pl.BlockSpec((NC, RB, F2 // 2), lambda i: (0, i, 0)),
        out_shape=jax.ShapeDtypeStruct((NC, N, F2 // 2), jnp.float32),
    )(s1, gs, d0, d1, Wmua, Wmub, Wlva, Wlvb, b1a, b1b)


def _tc3_body(s2_ref, g1s_ref, d0_ref, d1_ref, bmu_ref, blv_ref, eps_ref,
              z_ref, mu_ref, lv_ref):
    dinv = lax.rsqrt(d0_ref[...] + d1_ref[...] + 1.0)
    mu = (s2_ref[0] + g1s_ref[0]) * dinv + bmu_ref[...]
    lv = (s2_ref[1] + g1s_ref[1]) * dinv + blv_ref[...]
    mu_ref[...] = mu
    lv_ref[...] = lv
    z_ref[...] = mu + eps_ref[...] * jnp.exp(0.5 * lv)


def _tc3_call(s2, g1s, d0, d1, bmur, blvr, eps):
    return pl.pallas_call(
        _tc3_body,
        grid=(GRID,),
        in_specs=[
            pl.BlockSpec((NC, RB, F2 // 2), lambda i: (0, i, 0)),
            pl.BlockSpec((NC, RB, F2 // 2), lambda i: (0, i, 0)),
            pl.BlockSpec((RB, 1), lambda i: (i, 0)),
            pl.BlockSpec((RB, 1), lambda i: (i, 0)),
            pl.BlockSpec((1, LAT), lambda i: (0, 0)),
            pl.BlockSpec((1, LAT), lambda i: (0, 0)),
            pl.BlockSpec((RB, LAT), lambda i: (i, 0)),
        ],
        out_specs=[
            pl.BlockSpec((RB, LAT), lambda i: (i, 0)),
            pl.BlockSpec((RB, LAT), lambda i: (i, 0)),
            pl.BlockSpec((RB, LAT), lambda i: (i, 0)),
        ],
        out_shape=[
            jax.ShapeDtypeStruct((N, LAT), jnp.float32),
            jax.ShapeDtypeStruct((N, LAT), jnp.float32),
            jax.ShapeDtypeStruct((N, LAT), jnp.float32),
        ],
    )(s2, g1s, d0, d1, bmur, blvr, eps)


def kernel(user_x, item_x, edge_index, W1, b1, Wmu, bmu, Wlv, blv, eps):
    x = jnp.concatenate([user_x, item_x], axis=0)
    sd = edge_index.astype(jnp.int32).reshape(2, NCHT, CH)

    degp = _deg_call(sd)
    d0 = degp[0][:N, None]
    d1 = degp[1][:N, None]

    gs = _tc1_call(x, W1[:, :D // 2], W1[:, D // 2:], d0, d1)
    s1 = _scatter_call(D // 2, gs, sd)

    g1s = _tc2_call(s1, gs, d0, d1,
                    Wmu[:D // 2], Wmu[D // 2:], Wlv[:D // 2], Wlv[D // 2:],
                    b1[:D // 2].reshape(1, -1), b1[D // 2:].reshape(1, -1))
    s2 = _scatter_call(F2 // 2, g1s, sd, from_spmem=True)

    z, mu, lv = _tc3_call(s2, g1s, d0, d1,
                          bmu.reshape(1, LAT), blv.reshape(1, LAT), eps)
    return (z, mu, lv)


# speedup vs baseline: 1.1003x; 1.1003x over previous
import jax
import jax.numpy as jnp
from jax import lax
from jax.experimental import pallas as pl
from jax.experimental.pallas import tpu as pltpu
from jax.experimental.pallas import tpu_sc as plsc

N = 10000
D = 128
F2 = 64
LAT = 32
E = 320000
NC, NS = 2, 16
CH = 125
NCHT = E // CH
NCH = NCHT // NS
NACC = 10240
RPT = NACC // NS
ZR = 128
NBUF = 5
NWK = NC * NS
DEGC = NCHT // NWK
RB = 2000
GRID = N // RB


def _sc_mesh():
    return plsc.VectorSubcoreMesh(core_axis_name="c", subcore_axis_name="s")


def _deg_body(sd_hbm, out_hbm, dstv, onev, zbuf, acc, sem):
    c = lax.axis_index("c")
    s = lax.axis_index("s")
    wid = s * NC + c
    pltpu.sync_copy(sd_hbm.at[1, pl.ds(wid * DEGC, DEGC)], dstv)

    for i in range(7):
        onev[pl.ds(i * 16, 16)] = jnp.ones((16,), jnp.float32)
    onev[pl.ds(CH - 16, 16)] = jnp.ones((16,), jnp.float32)

    def _setzero(i, carry):
        zbuf[pl.ds(i * 16, 16)] = jnp.zeros((16,), jnp.float32)
        return carry

    lax.fori_loop(0, RPT // 16, _setzero, 0)
    pltpu.sync_copy(zbuf, acc.at[pl.ds(s * RPT, RPT)])
    plsc.subcore_barrier()

    def _chunk(j, carry):
        pltpu.async_copy(onev, acc.at[dstv.at[j]], sem, add=True)
        return carry

    lax.fori_loop(0, DEGC, _chunk, 0)

    def _drain(j, carry):
        pltpu.make_async_copy(onev, acc.at[dstv.at[0]], sem).wait()
        return carry

    lax.fori_loop(0, DEGC, _drain, 0)
    plsc.subcore_barrier()
    pltpu.sync_copy(acc.at[pl.ds(s * RPT, RPT)], out_hbm.at[c, pl.ds(s * RPT, RPT)])


def _deg_call(sd):
    f = pl.kernel(
        _deg_body,
        out_type=pltpu.HBM((NC, NACC), jnp.float32),
        mesh=_sc_mesh(),
        scratch_types=[
            pltpu.VMEM((DEGC, CH), jnp.int32),
            pltpu.VMEM((CH,), jnp.float32),
            pltpu.VMEM((RPT,), jnp.float32),
            pltpu.VMEM_SHARED((NACC,), jnp.float32),
            pltpu.SemaphoreType.DMA,
        ],
    )
    return f(sd)


def _make_scatter_body(H, from_spmem):
    def body(g_hbm, sd_hbm, out_hbm,
             srcv, dstv, rows, zbuf, acc, *rest):
        if from_spmem:
            gsp = rest[0]
            rest = rest[1:]
        gsem = list(rest[:NBUF])
        ssem = list(rest[NBUF:])
        c = lax.axis_index("c")
        s = lax.axis_index("s")
        pltpu.sync_copy(sd_hbm.at[0, pl.ds(s * NCH, NCH)], srcv)
        pltpu.sync_copy(sd_hbm.at[1, pl.ds(s * NCH, NCH)], dstv)

        if from_spmem:
            rpg = N // NS
            pltpu.sync_copy(g_hbm.at[c, pl.ds(s * rpg, rpg)],
                            gsp.at[pl.ds(s * rpg, rpg)])
            gref = gsp
        else:
            gref = g_hbm.at[c]

        K = NBUF // 2
        if not from_spmem:
            for b in range(K):
                pltpu.async_copy(gref.at[srcv.at[b]], rows.at[b], gsem[b])

        def _zrow(i, carry):
            for k in range(H // 16):
                zbuf[i, pl.ds(k * 16, 16)] = jnp.zeros((16,), jnp.float32)
            return carry

        lax.fori_loop(0, ZR, _zrow, 0)
        for r in range(RPT // ZR):
            pltpu.sync_copy(zbuf, acc.at[pl.ds(s * RPT + r * ZR, ZR)])
        plsc.subcore_barrier()

        for b in range(0 if from_spmem else K, NBUF):
            pltpu.async_copy(gref.at[srcv.at[b]], rows.at[b], gsem[b])

        def outer(j0, carry):
            for b in range(NBUF):
                j = j0 + b
                pltpu.make_async_copy(gref.at[srcv.at[0]], rows.at[b],
                                      gsem[b]).wait()
                pltpu.async_copy(rows.at[b], acc.at[dstv.at[j]],
                                 ssem[b], add=True)
            for b in range(NBUF):
                j = j0 + b
                pltpu.make_async_copy(rows.at[b], acc.at[dstv.at[0]],
                                      ssem[b]).wait()

                @pl.when(j + NBUF < NCH)
                def _fire(b=b, j=j):
                    pltpu.async_copy(gref.at[srcv.at[j + NBUF]],
                                     rows.at[b], gsem[b])
            return carry

        lax.fori_loop(0, NCH // NBUF, lambda i, cr: outer(i * NBUF, cr), 0)
        plsc.subcore_barrier()
        pltpu.sync_copy(acc.at[pl.ds(s * RPT, RPT)],
                        out_hbm.at[c, pl.ds(s * RPT, RPT)])

    return body


def _scatter_call(H, gs, sd, from_spmem=False):
    f = pl.kernel(
        _make_scatter_body(H, from_spmem),
        out_type=pltpu.HBM((NC, NACC, H), jnp.float32),
        mesh=_sc_mesh(),
        compiler_params=pltpu.CompilerParams(use_tc_tiling_on_sc=False),
        scratch_types=(
            [pltpu.VMEM((NCH, CH), jnp.int32),
             pltpu.VMEM((NCH, CH), jnp.int32),
             pltpu.VMEM((NBUF, CH, H), jnp.float32),
             pltpu.VMEM((ZR, H), jnp.float32),
             pltpu.VMEM_SHARED((NACC, H), jnp.float32)]
            + ([pltpu.VMEM_SHARED((N, H), jnp.float32)] if from_spmem else [])
            + [pltpu.SemaphoreType.DMA] * (2 * NBUF)
        ),
    )
    return f(gs, sd)


def _tc1_body(x_ref, wa_ref, wb_ref, d0_ref, d1_ref, gs_ref):
    dinv = lax.rsqrt(d0_ref[...] + d1_ref[...] + 1.0)
    x = x_ref[...]
    gs_ref[0] = jnp.dot(x, wa_ref[...],
                        preferred_element_type=jnp.float32) * dinv
    gs_ref[1] = jnp.dot(x, wb_ref[...],
                        preferred_element_type=jnp.float32) * dinv


def _tc1_call(x, W1a, W1b, d0, d1):
    return pl.pallas_call(
        _tc1_body,
        grid=(GRID,),
        in_specs=[
            pl.BlockSpec((RB, D), lambda i: (i, 0)),
            pl.BlockSpec((D, D // 2), lambda i: (0, 0)),
            pl.BlockSpec((D, D // 2), lambda i: (0, 0)),
            pl.BlockSpec((RB, 1), lambda i: (i, 0)),
            pl.BlockSpec((RB, 1), lambda i: (i, 0)),
        ],
        out_specs=pl.BlockSpec((NC, RB, D // 2), lambda i: (0, i, 0)),
        out_shape=jax.ShapeDtypeStruct((NC, N, D // 2), jnp.float32),
    )(x, W1a, W1b, d0, d1)


def _tc2_body(s1_ref, gs_ref, d0_ref, d1_ref, wmua_ref, wmub_ref,
              wlva_ref, wlvb_ref, b1a_ref, b1b_ref, g1s_ref):
    dinv = lax.rsqrt(d0_ref[...] + d1_ref[...] + 1.0)
    ha = jnp.maximum((s1_ref[0] + gs_ref[0]) * dinv + b1a_ref[...], 0.0)
    hb = jnp.maximum((s1_ref[1] + gs_ref[1]) * dinv + b1b_ref[...], 0.0)
    mm = jnp.dot(ha, wmua_ref[...], preferred_element_type=jnp.float32)
    mm += jnp.dot(hb, wmub_ref[...], preferred_element_type=jnp.float32)
    lv = jnp.dot(ha, wlva_ref[...], preferred_element_type=jnp.float32)
    lv += jnp.dot(hb, wlvb_ref[...], preferred_element_type=jnp.float32)
    g1s_ref[0] = mm * dinv
    g1s_ref[1] = lv * dinv


def _tc2_call(s1, gs, d0, d1, Wmua, Wmub, Wlva, Wlvb, b1a, b1b):
    return pl.pallas_call(
        _tc2_body,
        grid=(GRID,),
        in_specs=[
            pl.BlockSpec((NC, RB, D // 2), lambda i: (0, i, 0)),
            pl.BlockSpec((NC, RB, D // 2), lambda i: (0, i, 0)),
            pl.BlockSpec((RB, 1), lambda i: (i, 0)),
            pl.BlockSpec((RB, 1), lambda i: (i, 0)),
            pl.BlockSpec((D // 2, LAT), lambda i: (0, 0)),
            pl.BlockSpec((D // 2, LAT), lambda i: (0, 0)),
            pl.BlockSpec((D // 2, LAT), lambda i: (0, 0)),
            pl.BlockSpec((D // 2, LAT), lambda i: (0, 0)),
            pl.BlockSpec((1, D // 2), lambda i: (0, 0)),
            pl.BlockSpec((1, D // 2), lambda i: (0, 0)),
        ],
        out_specs=pl.BlockSpec((NC, RB, F2 // 2), lambda i: (0, i, 0)),
        out_shape=jax.ShapeDtypeStruct((NC, N, F2 // 2), jnp.float32),
    )(s1, gs, d0, d1, Wmua, Wmub, Wlva, Wlvb, b1a, b1b)


def _tc3_body(s2_ref, g1s_ref, d0_ref, d1_ref, bmu_ref, blv_ref, eps_ref,
              z_ref, mu_ref, lv_ref):
    dinv = lax.rsqrt(d0_ref[...] + d1_ref[...] + 1.0)
    mu = (s2_ref[0] + g1s_ref[0]) * dinv + bmu_ref[...]
    lv = (s2_ref[1] + g1s_ref[1]) * dinv + blv_ref[...]
    mu_ref[...] = mu
    lv_ref[...] = lv
    z_ref[...] = mu + eps_ref[...] * jnp.exp(0.5 * lv)


def _tc3_call(s2, g1s, d0, d1, bmur, blvr, eps):
    return pl.pallas_call(
        _tc3_body,
        grid=(GRID,),
        in_specs=[
            pl.BlockSpec((NC, RB, F2 // 2), lambda i: (0, i, 0)),
            pl.BlockSpec((NC, RB, F2 // 2), lambda i: (0, i, 0)),
            pl.BlockSpec((RB, 1), lambda i: (i, 0)),
            pl.BlockSpec((RB, 1), lambda i: (i, 0)),
            pl.BlockSpec((1, LAT), lambda i: (0, 0)),
            pl.BlockSpec((1, LAT), lambda i: (0, 0)),
            pl.BlockSpec((RB, LAT), lambda i: (i, 0)),
        ],
        out_specs=[
            pl.BlockSpec((RB, LAT), lambda i: (i, 0)),
            pl.BlockSpec((RB, LAT), lambda i: (i, 0)),
            pl.BlockSpec((RB, LAT), lambda i: (i, 0)),
        ],
        out_shape=[
            jax.ShapeDtypeStruct((N, LAT), jnp.float32),
            jax.ShapeDtypeStruct((N, LAT), jnp.float32),
            jax.ShapeDtypeStruct((N, LAT), jnp.float32),
        ],
    )(s2, g1s, d0, d1, bmur, blvr, eps)


def kernel(user_x, item_x, edge_index, W1, b1, Wmu, bmu, Wlv, blv, eps):
    x = jnp.concatenate([user_x, item_x], axis=0)
    sd = edge_index.astype(jnp.int32).reshape(2, NCHT, CH)

    degp = _deg_call(sd)
    d0 = degp[0][:N, None]
    d1 = degp[1][:N, None]

    gs = _tc1_call(x, W1[:, :D // 2], W1[:, D // 2:], d0, d1)
    s1 = _scatter_call(D // 2, gs, sd)

    g1s = _tc2_call(s1, gs, d0, d1,
                    Wmu[:D // 2], Wmu[D // 2:], Wlv[:D // 2], Wlv[D // 2:],
                    b1[:D // 2].reshape(1, -1), b1[D // 2:].reshape(1, -1))
    s2 = _scatter_call(F2 // 2, g1s, sd)

    z, mu, lv = _tc3_call(s2, g1s, d0, d1,
                          bmu.reshape(1, LAT), blv.reshape(1, LAT), eps)
    return (z, mu, lv)
